# baseline (speedup 1.0000x reference)
import functools

import jax
import jax.numpy as jnp
from jax import lax
from jax.experimental import pallas as pl
from jax.experimental.pallas import tpu as pltpu
from jax.experimental.pallas import tpu_sc as plsc

_NC = 2
_NS = 16
_NW = _NC * _NS
_L = 16
_CHUNK = 128


def _sc_body(nch, d, ctable, ptable, xent, hidx, tidx, pidx, atom_out,
             x_v, hidx_v, tidx_v, pidx_v, hids_v, tids_v,
             hrows, trows, prows, sem):
  rpw = nch * _CHUNK
  wid = lax.axis_index("s") * _NC + lax.axis_index("c")
  base = wid * rpw

  pltpu.sync_copy(xent, x_v)
  pltpu.sync_copy(hidx.at[wid], hidx_v)
  pltpu.sync_copy(tidx.at[wid], tidx_v)
  pltpu.sync_copy(pidx.at[wid], pidx_v)

  for j in range(nch):
    for i in range(_CHUNK // _L):
      s = pl.ds(i * _L, _L)
      hids_v[j, s] = plsc.load_gather(x_v, [hidx_v[j, s]])
      tids_v[j, s] = plsc.load_gather(x_v, [tidx_v[j, s]])

  copies = []
  for j in range(nch):
    r = pl.ds(j * _CHUNK, _CHUNK)
    copies.append(pltpu.async_copy(ctable.at[hids_v.at[j]], hrows.at[r], sem))
    copies.append(pltpu.async_copy(ctable.at[tids_v.at[j]], trows.at[r], sem))
    copies.append(pltpu.async_copy(ptable.at[pidx_v.at[j]], prows.at[r], sem))
  for c in copies:
    c.wait()

  def row_body(rr, carry):
    for cj in range(d // _L):
      s = pl.ds(cj * _L, _L)
      hrows[rr, s] = hrows[rr, s] * trows[rr, s] * prows[rr, s]
    return carry

  lax.fori_loop(0, rpw, row_body, 0)

  pltpu.sync_copy(hrows, atom_out.at[pl.ds(base, rpw)])


def _sc_gather_mul(ctable, ptable, xent, hidx, tidx, pidx, n, d):
  nch = (n // _NW) // _CHUNK
  rpw = nch * _CHUNK
  mesh = plsc.VectorSubcoreMesh(
      core_axis_name="c", subcore_axis_name="s",
      num_cores=_NC, num_subcores=_NS)
  m = xent.shape[0]
  f = pl.kernel(
      functools.partial(_sc_body, nch, d),
      out_type=jax.ShapeDtypeStruct((n, d), jnp.float32),
      mesh=mesh,
      compiler_params=pltpu.CompilerParams(
          needs_layout_passes=False, use_tc_tiling_on_sc=False),
      scratch_types=[
          pltpu.VMEM((m,), jnp.int32),
          pltpu.VMEM((nch, _CHUNK), jnp.int32),
          pltpu.VMEM((nch, _CHUNK), jnp.int32),
          pltpu.VMEM((nch, _CHUNK), jnp.int32),
          pltpu.VMEM((nch, _CHUNK), jnp.int32),
          pltpu.VMEM((nch, _CHUNK), jnp.int32),
          pltpu.VMEM((rpw, d), jnp.float32),
          pltpu.VMEM((rpw, d), jnp.float32),
          pltpu.VMEM((rpw, d), jnp.float32),
          pltpu.SemaphoreType.DMA,
      ],
  )
  return f(ctable, ptable, xent, hidx, tidx, pidx)


def _mm_body(atom_ref, w_ref, b_ref, o_ref):
  o_ref[...] = (
      jnp.dot(atom_ref[...], w_ref[...], preferred_element_type=jnp.float32)
      + b_ref[...])


def _out_proj(atom, w, b):
  n, d = atom.shape
  bm = 2048
  return pl.pallas_call(
      _mm_body,
      grid=(n // bm,),
      in_specs=[
          pl.BlockSpec((bm, d), lambda i: (i, 0)),
          pl.BlockSpec((d, d), lambda i: (0, 0)),
          pl.BlockSpec((1, d), lambda i: (0, 0)),
      ],
      out_specs=pl.BlockSpec((bm, d), lambda i: (i, 0)),
      out_shape=jax.ShapeDtypeStruct((n, d), jnp.float32),
  )(atom, w, b.reshape(1, d))


def kernel(constant_table, predicate_table, W_out, b_out, x_entity,
           pred_ids, const_idx):
  n = pred_ids.shape[0]
  d = constant_table.shape[1]
  nch = (n // _NW) // _CHUNK
  xent = x_entity.astype(jnp.int32)
  hidx = const_idx[:, 0].astype(jnp.int32).reshape(_NW, nch, _CHUNK)
  tidx = const_idx[:, 1].astype(jnp.int32).reshape(_NW, nch, _CHUNK)
  pidx = pred_ids.astype(jnp.int32).reshape(_NW, nch, _CHUNK)
  atom = _sc_gather_mul(constant_table, predicate_table, xent,
                        hidx, tidx, pidx, n, d)
  return _out_proj(atom, W_out, b_out)
